# baseline (speedup 1.0000x reference)
import functools

import jax
import jax.numpy as jnp
from jax import lax
from jax.experimental import pallas as pl
from jax.experimental.pallas import tpu as pltpu
from jax.experimental.pallas import tpu_sc as plsc

N_NODES = 9999
N_PAD = 10112
NUM_GRAPHS = 1111
NPG = 9
NC = 2
NS = 16
NW = NC * NS
CHUNK = 512
CPW = 20
EPW = CPW * CHUNK
E_PAD = NW * EPW
ROWS_PER_TILE = N_PAD // NS
DEG_F = 16


def _mesh():
    return plsc.VectorSubcoreMesh(core_axis_name="c", subcore_axis_name="s")


_SC_PARAMS = pltpu.CompilerParams(use_tc_tiling_on_sc=False)


def _sc_degree(dst_r, ones_col, zeros_col):
    @functools.partial(
        pl.kernel,
        out_type=jax.ShapeDtypeStruct((NC, N_PAD, DEG_F), jnp.float32),
        mesh=_mesh(),
        compiler_params=_SC_PARAMS,
        scratch_types=[
            pltpu.VMEM((CPW, CHUNK), jnp.int32),
            pltpu.VMEM((CHUNK, DEG_F), jnp.float32),
            pltpu.VMEM_SHARED((N_PAD, DEG_F), jnp.float32),
        ],
    )
    def deg_kernel(dst_hbm, ones_hbm, zeros_hbm, out_hbm, dst_v, ones_v, acc_sh):
        cid = lax.axis_index("c")
        sid = lax.axis_index("s")
        wid = sid * NC + cid
        pltpu.sync_copy(dst_hbm.at[wid], dst_v)
        pltpu.sync_copy(ones_hbm, ones_v)
        r0 = sid * ROWS_PER_TILE
        pltpu.sync_copy(zeros_hbm.at[pl.ds(r0, ROWS_PER_TILE)],
                        acc_sh.at[pl.ds(r0, ROWS_PER_TILE)])
        plsc.subcore_barrier()

        def body(j, carry):
            pltpu.sync_copy(ones_v, acc_sh.at[dst_v.at[j]], add=True)
            return carry

        lax.fori_loop(0, CPW, body, 0)
        plsc.subcore_barrier()
        pltpu.sync_copy(acc_sh.at[pl.ds(r0, ROWS_PER_TILE)],
                        out_hbm.at[cid, pl.ds(r0, ROWS_PER_TILE)])

    return deg_kernel(dst_r, ones_col, zeros_col)


def _sc_edge_agg(table, src_r, dst_r, feat):
    @functools.partial(
        pl.kernel,
        out_type=jax.ShapeDtypeStruct((NC, N_PAD, feat), jnp.float32),
        mesh=_mesh(),
        compiler_params=_SC_PARAMS,
        scratch_types=[
            pltpu.VMEM((CPW, CHUNK), jnp.int32),
            pltpu.VMEM((CPW, CHUNK), jnp.int32),
            pltpu.VMEM((CHUNK, feat), jnp.float32),
            pltpu.VMEM((CHUNK, feat), jnp.float32),
            pltpu.VMEM_SHARED((N_PAD, feat), jnp.float32),
            pltpu.SemaphoreType.DMA,
            pltpu.SemaphoreType.DMA,
        ],
    )
    def agg_kernel(tab_hbm, src_hbm, dst_hbm, out_hbm,
                   src_v, dst_v, buf0, buf1, acc_sh, sem0, sem1):
        cid = lax.axis_index("c")
        sid = lax.axis_index("s")
        wid = sid * NC + cid
        pltpu.sync_copy(src_hbm.at[wid], src_v)
        pltpu.sync_copy(dst_hbm.at[wid], dst_v)
        r0 = sid * ROWS_PER_TILE

        pltpu.sync_copy(tab_hbm.at[pl.ds(r0, ROWS_PER_TILE)],
                        acc_sh.at[pl.ds(r0, ROWS_PER_TILE)])

        plsc.subcore_barrier()

        pltpu.async_copy(tab_hbm.at[src_v.at[0]], buf0, sem0)

        def wait_gather(buf, sem):
            pltpu.make_async_copy(tab_hbm.at[pl.ds(0, CHUNK)], buf, sem).wait()

        def body(t, carry):
            j0 = 2 * t
            g1 = pltpu.async_copy(tab_hbm.at[src_v.at[j0 + 1]], buf1, sem1)
            wait_gather(buf0, sem0)
            pltpu.sync_copy(buf0, acc_sh.at[dst_v.at[j0]], add=True)

            @pl.when(t < CPW // 2 - 1)
            def _():
                pltpu.async_copy(tab_hbm.at[src_v.at[j0 + 2]], buf0, sem0)

            g1.wait()
            pltpu.sync_copy(buf1, acc_sh.at[dst_v.at[j0 + 1]], add=True)
            return carry

        lax.fori_loop(0, CPW // 2, body, 0)
        plsc.subcore_barrier()
        pltpu.sync_copy(acc_sh.at[pl.ds(r0, ROWS_PER_TILE)],
                        out_hbm.at[cid, pl.ds(r0, ROWS_PER_TILE)])

    return agg_kernel(table, src_r, dst_r)


def _tc_dense1(x_pad, deg_ab, w1):
    def body(x_ref, deg_ref, w_ref, scaled_ref, dinv_ref):
        deg = deg_ref[0, :, :1] + deg_ref[1, :, :1] + 1.0
        dinv = lax.rsqrt(deg)
        h = jnp.dot(x_ref[...], w_ref[...], preferred_element_type=jnp.float32)
        scaled_ref[...] = h * dinv
        dinv_ref[...] = dinv

    return pl.pallas_call(
        body,
        out_shape=(jax.ShapeDtypeStruct((N_PAD, 32), jnp.float32),
                   jax.ShapeDtypeStruct((N_PAD, 1), jnp.float32)),
    )(x_pad, deg_ab, w1)


def _tc_dense2(agg_ab, scaled1, dinv, b1, w2):
    def body(agg_ref, sc_ref, dinv_ref, b_ref, w_ref, out_ref):
        dinv = dinv_ref[...]
        tot = agg_ref[0] + agg_ref[1] - sc_ref[...]
        out1 = jnp.maximum(tot * dinv + b_ref[...], 0.0)
        h2 = jnp.dot(out1, w_ref[...], preferred_element_type=jnp.float32)
        out_ref[...] = h2 * dinv

    return pl.pallas_call(
        body,
        out_shape=jax.ShapeDtypeStruct((N_PAD, 64), jnp.float32),
    )(agg_ab, scaled1, dinv, b1, w2)


def _tc_dense3(agg_ab, scaled2, dinv, b2):
    def body(agg_ref, sc_ref, dinv_ref, b_ref, out_ref):
        tot = agg_ref[0] + agg_ref[1] - sc_ref[...]
        out_ref[...] = jnp.maximum(tot * dinv_ref[...] + b_ref[...], 0.0)

    return pl.pallas_call(
        body,
        out_shape=jax.ShapeDtypeStruct((N_PAD, 64), jnp.float32),
    )(agg_ab, scaled2, dinv, b2)


def _tc_heads(x3, wtop, wbot, bf1, wf2, bf2):
    def body(x3_ref, wtop_ref, wbot_ref, bf1_ref, wf2_ref, bf2_ref, out_ref):
        x3 = x3_ref[...]
        ctx = x3[:, 0:64]
        for k in range(1, NPG):
            ctx = ctx + x3[:, 64 * k:64 * k + 64]
        ctx = ctx * (1.0 / NPG)
        ctx_part = jnp.dot(ctx, wbot_ref[...], preferred_element_type=jnp.float32)
        qs = []
        for node in (0, 8):
            s = x3[:, 64 * node:64 * node + 64]
            pre = jnp.dot(s, wtop_ref[...], preferred_element_type=jnp.float32)
            pre = jnp.maximum(pre + ctx_part + bf1_ref[...], 0.0)
            q = jnp.dot(pre, wf2_ref[...], preferred_element_type=jnp.float32)
            qs.append(q + bf2_ref[...])
        out_ref[...] = jnp.concatenate(qs, axis=1)

    rows = x3.shape[0]
    return pl.pallas_call(
        body,
        out_shape=jax.ShapeDtypeStruct((rows, 2), jnp.float32),
    )(x3, wtop, wbot, bf1, wf2, bf2)


def kernel(x, edge_index, batch, W1, b1, W2, b2, Wf1, bf1, Wf2, bf2):
    del batch

    src = edge_index[0].astype(jnp.int32)
    dst = edge_index[1].astype(jnp.int32)
    n_edges = src.shape[0]
    pad = E_PAD - n_edges
    src_r = jnp.concatenate([src, jnp.zeros((pad,), jnp.int32)]).reshape(NW, CPW, CHUNK)
    dst_r = jnp.concatenate([dst, jnp.full((pad,), N_NODES, jnp.int32)]).reshape(NW, CPW, CHUNK)

    x_pad = jnp.concatenate([x, jnp.zeros((N_PAD - N_NODES, x.shape[1]), x.dtype)])
    ones_col = jnp.ones((CHUNK, DEG_F), jnp.float32)
    zeros_col = jnp.zeros((N_PAD, DEG_F), jnp.float32)

    deg_ab = _sc_degree(dst_r, ones_col, zeros_col)
    scaled1, dinv = _tc_dense1(x_pad, deg_ab, W1)

    agg1 = _sc_edge_agg(scaled1, src_r, dst_r, 32)
    scaled2 = _tc_dense2(agg1, scaled1, dinv, b1.reshape(1, 32), W2)

    agg2 = _sc_edge_agg(scaled2, src_r, dst_r, 64)
    out2 = _tc_dense3(agg2, scaled2, dinv, b2.reshape(1, 64))

    x3 = out2[:N_NODES].reshape(NUM_GRAPHS, NPG * 64)
    x3 = jnp.concatenate([x3, jnp.zeros((1, NPG * 64), jnp.float32)])
    q = _tc_heads(x3, Wf1[:64], Wf1[64:], bf1.reshape(1, 64), Wf2,
                  bf2.reshape(1, 1))
    return q[:NUM_GRAPHS]


# speedup vs baseline: 1.1137x; 1.1137x over previous
import functools

import jax
import jax.numpy as jnp
from jax import lax
from jax.experimental import pallas as pl
from jax.experimental.pallas import tpu as pltpu
from jax.experimental.pallas import tpu_sc as plsc

N_NODES = 9999
N_PAD = 10112
NUM_GRAPHS = 1111
NPG = 9
NC = 2
NS = 16
NW = NC * NS
CHUNK = 256
CPW = 40
NBUF = 4
EPW = CPW * CHUNK
E_PAD = NW * EPW
ROWS_PER_TILE = N_PAD // NS
DEG_F = 16


def _mesh():
    return plsc.VectorSubcoreMesh(core_axis_name="c", subcore_axis_name="s")


_SC_PARAMS = pltpu.CompilerParams(use_tc_tiling_on_sc=False)


def _sc_degree(dst_r, ones_col, zeros_col):
    @functools.partial(
        pl.kernel,
        out_type=jax.ShapeDtypeStruct((NC, N_PAD, DEG_F), jnp.float32),
        mesh=_mesh(),
        compiler_params=_SC_PARAMS,
        scratch_types=[
            pltpu.VMEM((CPW, CHUNK), jnp.int32),
            pltpu.VMEM((CHUNK, DEG_F), jnp.float32),
            pltpu.VMEM_SHARED((N_PAD, DEG_F), jnp.float32),
        ],
    )
    def deg_kernel(dst_hbm, ones_hbm, zeros_hbm, out_hbm, dst_v, ones_v, acc_sh):
        cid = lax.axis_index("c")
        sid = lax.axis_index("s")
        wid = sid * NC + cid
        pltpu.sync_copy(dst_hbm.at[wid], dst_v)
        pltpu.sync_copy(ones_hbm, ones_v)
        r0 = sid * ROWS_PER_TILE
        pltpu.sync_copy(zeros_hbm.at[pl.ds(r0, ROWS_PER_TILE)],
                        acc_sh.at[pl.ds(r0, ROWS_PER_TILE)])
        plsc.subcore_barrier()

        def body(j, carry):
            pltpu.sync_copy(ones_v, acc_sh.at[dst_v.at[j]], add=True)
            return carry

        lax.fori_loop(0, CPW, body, 0)
        plsc.subcore_barrier()
        pltpu.sync_copy(acc_sh.at[pl.ds(r0, ROWS_PER_TILE)],
                        out_hbm.at[cid, pl.ds(r0, ROWS_PER_TILE)])

    return deg_kernel(dst_r, ones_col, zeros_col)


def _sc_edge_agg(table, src_r, dst_r, zeros_tab, feat):
    @functools.partial(
        pl.kernel,
        out_type=jax.ShapeDtypeStruct((NC, N_PAD, feat), jnp.float32),
        mesh=_mesh(),
        compiler_params=_SC_PARAMS,
        scratch_types=[
            pltpu.VMEM((CPW, CHUNK), jnp.int32),
            pltpu.VMEM((CPW, CHUNK), jnp.int32),
            [pltpu.VMEM((CHUNK, feat), jnp.float32) for _ in range(NBUF)],
            [pltpu.SemaphoreType.DMA for _ in range(NBUF)],
            [pltpu.SemaphoreType.DMA for _ in range(NBUF)],
            pltpu.VMEM_SHARED((N_PAD, feat), jnp.float32),
        ],
    )
    def agg_kernel(tab_hbm, src_hbm, dst_hbm, zeros_hbm, out_hbm,
                   src_v, dst_v, bufs, gsems, ssems, acc_sh):
        cid = lax.axis_index("c")
        sid = lax.axis_index("s")
        wid = sid * NC + cid
        pltpu.sync_copy(src_hbm.at[wid], src_v)
        pltpu.sync_copy(dst_hbm.at[wid], dst_v)
        r0 = sid * ROWS_PER_TILE

        @pl.when(cid == 0)
        def _():
            pltpu.sync_copy(tab_hbm.at[pl.ds(r0, ROWS_PER_TILE)],
                            acc_sh.at[pl.ds(r0, ROWS_PER_TILE)])

        @pl.when(cid != 0)
        def _():
            pltpu.sync_copy(zeros_hbm.at[pl.ds(r0, ROWS_PER_TILE)],
                            acc_sh.at[pl.ds(r0, ROWS_PER_TILE)])

        plsc.subcore_barrier()

        def wait_gather(b):
            pltpu.make_async_copy(tab_hbm.at[pl.ds(0, CHUNK)], bufs[b],
                                  gsems[b]).wait()

        def wait_scatter(b):
            pltpu.make_async_copy(bufs[b], acc_sh.at[dst_v.at[0]],
                                  ssems[b]).wait()

        def body(t, carry):
            j0 = NBUF * t
            for b in range(NBUF):
                @pl.when(t > 0)
                def _(b=b):
                    wait_scatter(b)
                pltpu.async_copy(tab_hbm.at[src_v.at[j0 + b]], bufs[b],
                                 gsems[b])
            for b in range(NBUF):
                wait_gather(b)
                pltpu.async_copy(bufs[b], acc_sh.at[dst_v.at[j0 + b]],
                                 ssems[b], add=True)
            return carry

        lax.fori_loop(0, CPW // NBUF, body, 0)
        for b in range(NBUF):
            wait_scatter(b)
        plsc.subcore_barrier()
        pltpu.sync_copy(acc_sh.at[pl.ds(r0, ROWS_PER_TILE)],
                        out_hbm.at[cid, pl.ds(r0, ROWS_PER_TILE)])

    return agg_kernel(table, src_r, dst_r, zeros_tab)


def _tc_dense1(x_pad, deg_ab, w1):
    def body(x_ref, deg_ref, w_ref, scaled_ref, dinv_ref):
        deg = deg_ref[0, :, :1] + deg_ref[1, :, :1] + 1.0
        dinv = lax.rsqrt(deg)
        h = jnp.dot(x_ref[...], w_ref[...], preferred_element_type=jnp.float32)
        scaled_ref[...] = h * dinv
        dinv_ref[...] = dinv

    return pl.pallas_call(
        body,
        out_shape=(jax.ShapeDtypeStruct((N_PAD, 32), jnp.float32),
                   jax.ShapeDtypeStruct((N_PAD, 1), jnp.float32)),
    )(x_pad, deg_ab, w1)


def _tc_dense2(agg_ab, dinv, b1, w2):
    def body(agg_ref, dinv_ref, b_ref, w_ref, out_ref):
        dinv = dinv_ref[...]
        tot = agg_ref[0] + agg_ref[1]
        out1 = jnp.maximum(tot * dinv + b_ref[...], 0.0)
        h2 = jnp.dot(out1, w_ref[...], preferred_element_type=jnp.float32)
        out_ref[...] = h2 * dinv

    return pl.pallas_call(
        body,
        out_shape=jax.ShapeDtypeStruct((N_PAD, 64), jnp.float32),
    )(agg_ab, dinv, b1, w2)


def _tc_dense3(agg_ab, dinv, b2):
    def body(agg_ref, dinv_ref, b_ref, out_ref):
        tot = agg_ref[0] + agg_ref[1]
        out_ref[...] = jnp.maximum(tot * dinv_ref[...] + b_ref[...], 0.0)

    return pl.pallas_call(
        body,
        out_shape=jax.ShapeDtypeStruct((N_PAD, 64), jnp.float32),
    )(agg_ab, dinv, b2)


def _tc_heads(x3, wtop, wbot, bf1, wf2, bf2):
    def body(x3_ref, wtop_ref, wbot_ref, bf1_ref, wf2_ref, bf2_ref, out_ref):
        x3 = x3_ref[...]
        ctx = x3[:, 0:64]
        for k in range(1, NPG):
            ctx = ctx + x3[:, 64 * k:64 * k + 64]
        ctx = ctx * (1.0 / NPG)
        ctx_part = jnp.dot(ctx, wbot_ref[...], preferred_element_type=jnp.float32)
        qs = []
        for node in (0, 8):
            s = x3[:, 64 * node:64 * node + 64]
            pre = jnp.dot(s, wtop_ref[...], preferred_element_type=jnp.float32)
            pre = jnp.maximum(pre + ctx_part + bf1_ref[...], 0.0)
            q = jnp.dot(pre, wf2_ref[...], preferred_element_type=jnp.float32)
            qs.append(q + bf2_ref[...])
        out_ref[...] = jnp.concatenate(qs, axis=1)

    rows = x3.shape[0]
    return pl.pallas_call(
        body,
        out_shape=jax.ShapeDtypeStruct((rows, 2), jnp.float32),
    )(x3, wtop, wbot, bf1, wf2, bf2)


def kernel(x, edge_index, batch, W1, b1, W2, b2, Wf1, bf1, Wf2, bf2):
    del batch

    src = edge_index[0].astype(jnp.int32)
    dst = edge_index[1].astype(jnp.int32)
    n_edges = src.shape[0]
    pad = E_PAD - n_edges
    src_r = jnp.concatenate([src, jnp.zeros((pad,), jnp.int32)]).reshape(NW, CPW, CHUNK)
    dst_r = jnp.concatenate([dst, jnp.full((pad,), N_NODES, jnp.int32)]).reshape(NW, CPW, CHUNK)

    x_pad = jnp.concatenate([x, jnp.zeros((N_PAD - N_NODES, x.shape[1]), x.dtype)])
    ones_col = jnp.ones((CHUNK, DEG_F), jnp.float32)
    zeros_col = jnp.zeros((N_PAD, DEG_F), jnp.float32)
    zeros32 = jnp.zeros((N_PAD, 32), jnp.float32)
    zeros64 = jnp.zeros((N_PAD, 64), jnp.float32)

    deg_ab = _sc_degree(dst_r, ones_col, zeros_col)
    scaled1, dinv = _tc_dense1(x_pad, deg_ab, W1)

    agg1 = _sc_edge_agg(scaled1, src_r, dst_r, zeros32, 32)
    scaled2 = _tc_dense2(agg1, dinv, b1.reshape(1, 32), W2)

    agg2 = _sc_edge_agg(scaled2, src_r, dst_r, zeros64, 64)
    out2 = _tc_dense3(agg2, dinv, b2.reshape(1, 64))

    x3 = out2[:N_NODES].reshape(NUM_GRAPHS, NPG * 64)
    x3 = jnp.concatenate([x3, jnp.zeros((1, NPG * 64), jnp.float32)])
    q = _tc_heads(x3, Wf1[:64], Wf1[64:], bf1.reshape(1, 64), Wf2,
                  bf2.reshape(1, 1))
    return q[:NUM_GRAPHS]


# speedup vs baseline: 1.2823x; 1.1514x over previous
import functools

import jax
import jax.numpy as jnp
from jax import lax
from jax.experimental import pallas as pl
from jax.experimental.pallas import tpu as pltpu
from jax.experimental.pallas import tpu_sc as plsc

N_NODES = 9999
N_PAD = 10112
NUM_GRAPHS = 1111
NPG = 9
NC = 2
NS = 16
NW = NC * NS
CHUNK = 256
CPW = 40
NBUF = 4
EPW = CPW * CHUNK
E_PAD = NW * EPW
ROWS_PER_TILE = N_PAD // NS
DEG_F = 16


def _mesh():
    return plsc.VectorSubcoreMesh(core_axis_name="c", subcore_axis_name="s")


_SC_PARAMS = pltpu.CompilerParams(use_tc_tiling_on_sc=False)


def _sc_degree(dst_r, ones_col, zeros_col):
    @functools.partial(
        pl.kernel,
        out_type=jax.ShapeDtypeStruct((NC, N_PAD, DEG_F), jnp.float32),
        mesh=_mesh(),
        compiler_params=_SC_PARAMS,
        scratch_types=[
            pltpu.VMEM((CPW, CHUNK), jnp.int32),
            pltpu.VMEM((CHUNK, DEG_F), jnp.float32),
            pltpu.VMEM_SHARED((N_PAD, DEG_F), jnp.float32),
        ],
    )
    def deg_kernel(dst_hbm, ones_hbm, zeros_hbm, out_hbm, dst_v, ones_v, acc_sh):
        cid = lax.axis_index("c")
        sid = lax.axis_index("s")
        wid = sid * NC + cid
        pltpu.sync_copy(dst_hbm.at[wid], dst_v)
        pltpu.sync_copy(ones_hbm, ones_v)
        r0 = sid * ROWS_PER_TILE
        pltpu.sync_copy(zeros_hbm.at[pl.ds(r0, ROWS_PER_TILE)],
                        acc_sh.at[pl.ds(r0, ROWS_PER_TILE)])
        plsc.subcore_barrier()

        def body(j, carry):
            pltpu.sync_copy(ones_v, acc_sh.at[dst_v.at[j]], add=True)
            return carry

        lax.fori_loop(0, CPW, body, 0)
        plsc.subcore_barrier()
        pltpu.sync_copy(acc_sh.at[pl.ds(r0, ROWS_PER_TILE)],
                        out_hbm.at[cid, pl.ds(r0, ROWS_PER_TILE)])

    return deg_kernel(dst_r, ones_col, zeros_col)


def _sc_edge_agg(table, src_r, dst_r, zeros_tab, feat, stage_table=False):
    scratch = [
        pltpu.VMEM((CPW, CHUNK), jnp.int32),
        pltpu.VMEM((CPW, CHUNK), jnp.int32),
        [pltpu.VMEM((CHUNK, feat), jnp.float32) for _ in range(NBUF)],
        [pltpu.SemaphoreType.DMA for _ in range(NBUF)],
        [pltpu.SemaphoreType.DMA for _ in range(NBUF)],
        pltpu.VMEM_SHARED((N_PAD, feat), jnp.float32),
    ]
    if stage_table:
        scratch.append(pltpu.VMEM_SHARED((N_PAD, feat), jnp.float32))

    @functools.partial(
        pl.kernel,
        out_type=jax.ShapeDtypeStruct((NC, N_PAD, feat), jnp.float32),
        mesh=_mesh(),
        compiler_params=_SC_PARAMS,
        scratch_types=scratch,
    )
    def agg_kernel(tab_hbm, src_hbm, dst_hbm, zeros_hbm, out_hbm,
                   src_v, dst_v, bufs, gsems, ssems, acc_sh, *maybe_tab):
        tab_ref = maybe_tab[0] if stage_table else tab_hbm
        cid = lax.axis_index("c")
        sid = lax.axis_index("s")
        wid = sid * NC + cid
        pltpu.sync_copy(src_hbm.at[wid], src_v)
        pltpu.sync_copy(dst_hbm.at[wid], dst_v)
        r0 = sid * ROWS_PER_TILE

        if stage_table:
            pltpu.sync_copy(tab_hbm.at[pl.ds(r0, ROWS_PER_TILE)],
                            tab_ref.at[pl.ds(r0, ROWS_PER_TILE)])

        @pl.when(cid == 0)
        def _():
            pltpu.sync_copy(tab_hbm.at[pl.ds(r0, ROWS_PER_TILE)],
                            acc_sh.at[pl.ds(r0, ROWS_PER_TILE)])

        @pl.when(cid != 0)
        def _():
            pltpu.sync_copy(zeros_hbm.at[pl.ds(r0, ROWS_PER_TILE)],
                            acc_sh.at[pl.ds(r0, ROWS_PER_TILE)])

        plsc.subcore_barrier()

        def wait_gather(b):
            pltpu.make_async_copy(tab_hbm.at[pl.ds(0, CHUNK)], bufs[b],
                                  gsems[b]).wait()

        def wait_scatter(b):
            pltpu.make_async_copy(bufs[b], acc_sh.at[dst_v.at[0]],
                                  ssems[b]).wait()

        def body(t, carry):
            j0 = NBUF * t
            for b in range(NBUF):
                @pl.when(t > 0)
                def _(b=b):
                    wait_scatter(b)
                pltpu.async_copy(tab_ref.at[src_v.at[j0 + b]], bufs[b],
                                 gsems[b])
            for b in range(NBUF):
                wait_gather(b)
                pltpu.async_copy(bufs[b], acc_sh.at[dst_v.at[j0 + b]],
                                 ssems[b], add=True)
            return carry

        lax.fori_loop(0, CPW // NBUF, body, 0)
        for b in range(NBUF):
            wait_scatter(b)
        plsc.subcore_barrier()
        pltpu.sync_copy(acc_sh.at[pl.ds(r0, ROWS_PER_TILE)],
                        out_hbm.at[cid, pl.ds(r0, ROWS_PER_TILE)])

    return agg_kernel(table, src_r, dst_r, zeros_tab)


def _tc_dense1(x_pad, deg_ab, w1):
    def body(x_ref, deg_ref, w_ref, scaled_ref, dinv_ref):
        deg = deg_ref[0, :, :1] + deg_ref[1, :, :1] + 1.0
        dinv = lax.rsqrt(deg)
        h = jnp.dot(x_ref[...], w_ref[...], preferred_element_type=jnp.float32)
        scaled_ref[...] = h * dinv
        dinv_ref[...] = dinv

    return pl.pallas_call(
        body,
        out_shape=(jax.ShapeDtypeStruct((N_PAD, 32), jnp.float32),
                   jax.ShapeDtypeStruct((N_PAD, 1), jnp.float32)),
    )(x_pad, deg_ab, w1)


def _tc_dense2(agg_ab, dinv, b1, w2):
    def body(agg_ref, dinv_ref, b_ref, w_ref, out_ref):
        dinv = dinv_ref[...]
        tot = agg_ref[0] + agg_ref[1]
        out1 = jnp.maximum(tot * dinv + b_ref[...], 0.0)
        h2 = jnp.dot(out1, w_ref[...], preferred_element_type=jnp.float32)
        out_ref[...] = h2 * dinv

    return pl.pallas_call(
        body,
        out_shape=jax.ShapeDtypeStruct((N_PAD, 64), jnp.float32),
    )(agg_ab, dinv, b1, w2)


def _tc_dense3(agg_ab, dinv, b2):
    def body(agg_ref, dinv_ref, b_ref, out_ref):
        tot = agg_ref[0] + agg_ref[1]
        out_ref[...] = jnp.maximum(tot * dinv_ref[...] + b_ref[...], 0.0)

    return pl.pallas_call(
        body,
        out_shape=jax.ShapeDtypeStruct((N_PAD, 64), jnp.float32),
    )(agg_ab, dinv, b2)


def _tc_heads(x3, wtop, wbot, bf1, wf2, bf2):
    def body(x3_ref, wtop_ref, wbot_ref, bf1_ref, wf2_ref, bf2_ref, out_ref):
        x3 = x3_ref[...]
        ctx = x3[:, 0:64]
        for k in range(1, NPG):
            ctx = ctx + x3[:, 64 * k:64 * k + 64]
        ctx = ctx * (1.0 / NPG)
        ctx_part = jnp.dot(ctx, wbot_ref[...], preferred_element_type=jnp.float32)
        qs = []
        for node in (0, 8):
            s = x3[:, 64 * node:64 * node + 64]
            pre = jnp.dot(s, wtop_ref[...], preferred_element_type=jnp.float32)
            pre = jnp.maximum(pre + ctx_part + bf1_ref[...], 0.0)
            q = jnp.dot(pre, wf2_ref[...], preferred_element_type=jnp.float32)
            qs.append(q + bf2_ref[...])
        out_ref[...] = jnp.concatenate(qs, axis=1)

    rows = x3.shape[0]
    return pl.pallas_call(
        body,
        out_shape=jax.ShapeDtypeStruct((rows, 2), jnp.float32),
    )(x3, wtop, wbot, bf1, wf2, bf2)


def kernel(x, edge_index, batch, W1, b1, W2, b2, Wf1, bf1, Wf2, bf2):
    del batch

    src = edge_index[0].astype(jnp.int32)
    dst = edge_index[1].astype(jnp.int32)
    n_edges = src.shape[0]
    pad = E_PAD - n_edges
    src_r = jnp.concatenate([src, jnp.zeros((pad,), jnp.int32)]).reshape(NW, CPW, CHUNK)
    dst_r = jnp.concatenate([dst, jnp.full((pad,), N_NODES, jnp.int32)]).reshape(NW, CPW, CHUNK)

    x_pad = jnp.concatenate([x, jnp.zeros((N_PAD - N_NODES, x.shape[1]), x.dtype)])
    ones_col = jnp.ones((CHUNK, DEG_F), jnp.float32)
    zeros_col = jnp.zeros((N_PAD, DEG_F), jnp.float32)
    zeros32 = jnp.zeros((N_PAD, 32), jnp.float32)
    zeros64 = jnp.zeros((N_PAD, 64), jnp.float32)

    deg_ab = _sc_degree(dst_r, ones_col, zeros_col)
    scaled1, dinv = _tc_dense1(x_pad, deg_ab, W1)

    agg1 = _sc_edge_agg(scaled1, src_r, dst_r, zeros32, 32, stage_table=True)
    scaled2 = _tc_dense2(agg1, dinv, b1.reshape(1, 32), W2)

    agg2 = _sc_edge_agg(scaled2, src_r, dst_r, zeros64, 64)
    out2 = _tc_dense3(agg2, dinv, b2.reshape(1, 64))

    x3 = out2[:N_NODES].reshape(NUM_GRAPHS, NPG * 64)
    x3 = jnp.concatenate([x3, jnp.zeros((1, NPG * 64), jnp.float32)])
    q = _tc_heads(x3, Wf1[:64], Wf1[64:], bf1.reshape(1, 64), Wf2,
                  bf2.reshape(1, 1))
    return q[:NUM_GRAPHS]


# speedup vs baseline: 1.7123x; 1.3354x over previous
import functools

import jax
import jax.numpy as jnp
from jax import lax
from jax.experimental import pallas as pl
from jax.experimental.pallas import tpu as pltpu
from jax.experimental.pallas import tpu_sc as plsc

N_NODES = 9999
N_PAD = 10112
NUM_GRAPHS = 1111
NPG = 9
NC = 2
NS = 16
NW = NC * NS
CHUNK = 256
CPW = 40
NBUF = 4
EPW = CPW * CHUNK
E_PAD = NW * EPW
ROWS_PER_TILE = N_PAD // NS
DEG_F = 16


def _mesh():
    return plsc.VectorSubcoreMesh(core_axis_name="c", subcore_axis_name="s")


_SC_PARAMS = pltpu.CompilerParams(use_tc_tiling_on_sc=False)


def _sc_degree(dst_r, ones_col, zeros_col):
    @functools.partial(
        pl.kernel,
        out_type=jax.ShapeDtypeStruct((NC, N_PAD, DEG_F), jnp.float32),
        mesh=_mesh(),
        compiler_params=_SC_PARAMS,
        scratch_types=[
            pltpu.VMEM((CPW, CHUNK), jnp.int32),
            pltpu.VMEM((CHUNK, DEG_F), jnp.float32),
            pltpu.VMEM_SHARED((N_PAD, DEG_F), jnp.float32),
        ],
    )
    def deg_kernel(dst_hbm, ones_hbm, zeros_hbm, out_hbm, dst_v, ones_v, acc_sh):
        cid = lax.axis_index("c")
        sid = lax.axis_index("s")
        wid = sid * NC + cid
        pltpu.sync_copy(dst_hbm.at[wid], dst_v)
        pltpu.sync_copy(ones_hbm, ones_v)
        r0 = sid * ROWS_PER_TILE
        pltpu.sync_copy(zeros_hbm.at[pl.ds(r0, ROWS_PER_TILE)],
                        acc_sh.at[pl.ds(r0, ROWS_PER_TILE)])
        plsc.subcore_barrier()

        def body(j, carry):
            pltpu.sync_copy(ones_v, acc_sh.at[dst_v.at[j]], add=True)
            return carry

        lax.fori_loop(0, CPW, body, 0)
        plsc.subcore_barrier()
        pltpu.sync_copy(acc_sh.at[pl.ds(r0, ROWS_PER_TILE)],
                        out_hbm.at[cid, pl.ds(r0, ROWS_PER_TILE)])

    return deg_kernel(dst_r, ones_col, zeros_col)


def _sc_edge_agg(table, src_r, dst_r, zeros_tab, feat, stage_table=False):
    scratch = [
        pltpu.VMEM((CPW, CHUNK), jnp.int32),
        pltpu.VMEM((CPW, CHUNK), jnp.int32),
        [pltpu.VMEM((CHUNK, feat), jnp.float32) for _ in range(NBUF)],
        [pltpu.SemaphoreType.DMA for _ in range(NBUF)],
        [pltpu.SemaphoreType.DMA for _ in range(NBUF)],
        pltpu.VMEM_SHARED((N_PAD, feat), jnp.float32),
    ]
    if stage_table:
        scratch.append(pltpu.VMEM_SHARED((N_PAD, feat), jnp.float32))

    @functools.partial(
        pl.kernel,
        out_type=jax.ShapeDtypeStruct((NC, N_PAD, feat), jnp.float32),
        mesh=_mesh(),
        compiler_params=_SC_PARAMS,
        scratch_types=scratch,
    )
    def agg_kernel(tab_hbm, src_hbm, dst_hbm, zeros_hbm, out_hbm,
                   src_v, dst_v, bufs, gsems, ssems, acc_sh, *maybe_tab):
        tab_ref = maybe_tab[0] if stage_table else tab_hbm
        cid = lax.axis_index("c")
        sid = lax.axis_index("s")
        wid = sid * NC + cid
        pltpu.sync_copy(src_hbm.at[wid], src_v)
        pltpu.sync_copy(dst_hbm.at[wid], dst_v)
        r0 = sid * ROWS_PER_TILE

        if stage_table:
            pltpu.sync_copy(tab_hbm.at[pl.ds(r0, ROWS_PER_TILE)],
                            tab_ref.at[pl.ds(r0, ROWS_PER_TILE)])

        @pl.when(cid == 0)
        def _():
            pltpu.sync_copy(tab_hbm.at[pl.ds(r0, ROWS_PER_TILE)],
                            acc_sh.at[pl.ds(r0, ROWS_PER_TILE)])

        @pl.when(cid != 0)
        def _():
            pltpu.sync_copy(zeros_hbm.at[pl.ds(r0, ROWS_PER_TILE)],
                            acc_sh.at[pl.ds(r0, ROWS_PER_TILE)])

        plsc.subcore_barrier()

        def wait_gather(b):
            pltpu.make_async_copy(tab_hbm.at[pl.ds(0, CHUNK)], bufs[b],
                                  gsems[b]).wait()

        def wait_scatter(b):
            pltpu.make_async_copy(bufs[b], acc_sh.at[dst_v.at[0]],
                                  ssems[b]).wait()

        def body(t, carry):
            j0 = NBUF * t
            for b in range(NBUF):
                @pl.when(t > 0)
                def _(b=b):
                    wait_scatter(b)
                pltpu.async_copy(tab_ref.at[src_v.at[j0 + b]], bufs[b],
                                 gsems[b])
            for b in range(NBUF):
                wait_gather(b)
                pltpu.async_copy(bufs[b], acc_sh.at[dst_v.at[j0 + b]],
                                 ssems[b], add=True)
            return carry

        lax.fori_loop(0, CPW // NBUF, body, 0)
        for b in range(NBUF):
            wait_scatter(b)
        plsc.subcore_barrier()
        pltpu.sync_copy(acc_sh.at[pl.ds(r0, ROWS_PER_TILE)],
                        out_hbm.at[cid, pl.ds(r0, ROWS_PER_TILE)])

    return agg_kernel(table, src_r, dst_r, zeros_tab)


def _tc_dense1(x_pad, deg_ab, w1):
    def body(x_ref, deg_ref, w_ref, scaled_ref, dinv_ref):
        deg = deg_ref[0, :, :1] + deg_ref[1, :, :1] + 1.0
        dinv = lax.rsqrt(deg)
        h = jnp.dot(x_ref[...], w_ref[...], preferred_element_type=jnp.float32)
        scaled_ref[...] = h * dinv
        dinv_ref[...] = dinv

    return pl.pallas_call(
        body,
        out_shape=(jax.ShapeDtypeStruct((N_PAD, 32), jnp.float32),
                   jax.ShapeDtypeStruct((N_PAD, 1), jnp.float32)),
    )(x_pad, deg_ab, w1)


def _tc_dense2(agg_ab, dinv, b1, w2):
    def body(agg_ref, dinv_ref, b_ref, w_ref, outa_ref, outb_ref):
        dinv = dinv_ref[...]
        tot = agg_ref[0] + agg_ref[1]
        out1 = jnp.maximum(tot * dinv + b_ref[...], 0.0)
        h2 = jnp.dot(out1, w_ref[...], preferred_element_type=jnp.float32)
        scaled2 = h2 * dinv
        outa_ref[...] = scaled2[:, :32]
        outb_ref[...] = scaled2[:, 32:]

    return pl.pallas_call(
        body,
        out_shape=(jax.ShapeDtypeStruct((N_PAD, 32), jnp.float32),
                   jax.ShapeDtypeStruct((N_PAD, 32), jnp.float32)),
    )(agg_ab, dinv, b1, w2)


def _tc_dense3(agg_a, agg_b, dinv, b2):
    def body(agga_ref, aggb_ref, dinv_ref, b_ref, out_ref):
        tot = jnp.concatenate(
            [agga_ref[0] + agga_ref[1], aggb_ref[0] + aggb_ref[1]], axis=1)
        out_ref[...] = jnp.maximum(tot * dinv_ref[...] + b_ref[...], 0.0)

    return pl.pallas_call(
        body,
        out_shape=jax.ShapeDtypeStruct((N_PAD, 64), jnp.float32),
    )(agg_a, agg_b, dinv, b2)


def _tc_heads(x3, wtop, wbot, bf1, wf2, bf2):
    def body(x3_ref, wtop_ref, wbot_ref, bf1_ref, wf2_ref, bf2_ref, out_ref):
        x3 = x3_ref[...]
        ctx = x3[:, 0:64]
        for k in range(1, NPG):
            ctx = ctx + x3[:, 64 * k:64 * k + 64]
        ctx = ctx * (1.0 / NPG)
        ctx_part = jnp.dot(ctx, wbot_ref[...], preferred_element_type=jnp.float32)
        qs = []
        for node in (0, 8):
            s = x3[:, 64 * node:64 * node + 64]
            pre = jnp.dot(s, wtop_ref[...], preferred_element_type=jnp.float32)
            pre = jnp.maximum(pre + ctx_part + bf1_ref[...], 0.0)
            q = jnp.dot(pre, wf2_ref[...], preferred_element_type=jnp.float32)
            qs.append(q + bf2_ref[...])
        out_ref[...] = jnp.concatenate(qs, axis=1)

    rows = x3.shape[0]
    return pl.pallas_call(
        body,
        out_shape=jax.ShapeDtypeStruct((rows, 2), jnp.float32),
    )(x3, wtop, wbot, bf1, wf2, bf2)


def kernel(x, edge_index, batch, W1, b1, W2, b2, Wf1, bf1, Wf2, bf2):
    del batch

    src = edge_index[0].astype(jnp.int32)
    dst = edge_index[1].astype(jnp.int32)
    n_edges = src.shape[0]
    pad = E_PAD - n_edges
    src_r = jnp.concatenate([src, jnp.zeros((pad,), jnp.int32)]).reshape(NW, CPW, CHUNK)
    dst_r = jnp.concatenate([dst, jnp.full((pad,), N_NODES, jnp.int32)]).reshape(NW, CPW, CHUNK)

    x_pad = jnp.concatenate([x, jnp.zeros((N_PAD - N_NODES, x.shape[1]), x.dtype)])
    ones_col = jnp.ones((CHUNK, DEG_F), jnp.float32)
    zeros_col = jnp.zeros((N_PAD, DEG_F), jnp.float32)
    zeros32 = jnp.zeros((N_PAD, 32), jnp.float32)

    deg_ab = _sc_degree(dst_r, ones_col, zeros_col)
    scaled1, dinv = _tc_dense1(x_pad, deg_ab, W1)

    agg1 = _sc_edge_agg(scaled1, src_r, dst_r, zeros32, 32, stage_table=True)
    scaled2a, scaled2b = _tc_dense2(agg1, dinv, b1.reshape(1, 32), W2)

    agg2a = _sc_edge_agg(scaled2a, src_r, dst_r, zeros32, 32, stage_table=True)
    agg2b = _sc_edge_agg(scaled2b, src_r, dst_r, zeros32, 32, stage_table=True)
    out2 = _tc_dense3(agg2a, agg2b, dinv, b2.reshape(1, 64))

    x3 = out2[:N_NODES].reshape(NUM_GRAPHS, NPG * 64)
    x3 = jnp.concatenate([x3, jnp.zeros((1, NPG * 64), jnp.float32)])
    q = _tc_heads(x3, Wf1[:64], Wf1[64:], bf1.reshape(1, 64), Wf2,
                  bf2.reshape(1, 1))
    return q[:NUM_GRAPHS]


# speedup vs baseline: 1.7675x; 1.0322x over previous
import functools

import jax
import jax.numpy as jnp
from jax import lax
from jax.experimental import pallas as pl
from jax.experimental.pallas import tpu as pltpu
from jax.experimental.pallas import tpu_sc as plsc

N_NODES = 9999
N_PAD = 10112
NUM_GRAPHS = 1111
NPG = 9
NC = 2
NS = 16
NW = NC * NS
CHUNK = 256
CPW = 40
NBUF = 4
EPW = CPW * CHUNK
E_PAD = NW * EPW
CPT2 = CPW * NC
ROWS_PER_TILE = N_PAD // NS
DEG_F = 16


def _mesh():
    return plsc.VectorSubcoreMesh(core_axis_name="c", subcore_axis_name="s")


_SC_PARAMS = pltpu.CompilerParams(use_tc_tiling_on_sc=False)


def _sc_degree(dst_r, ones_col, zeros_col):
    @functools.partial(
        pl.kernel,
        out_type=jax.ShapeDtypeStruct((NC, N_PAD, DEG_F), jnp.float32),
        mesh=_mesh(),
        compiler_params=_SC_PARAMS,
        scratch_types=[
            pltpu.VMEM((CPW, CHUNK), jnp.int32),
            pltpu.VMEM((CHUNK, DEG_F), jnp.float32),
            pltpu.VMEM_SHARED((N_PAD, DEG_F), jnp.float32),
        ],
    )
    def deg_kernel(dst_hbm, ones_hbm, zeros_hbm, out_hbm, dst_v, ones_v, acc_sh):
        cid = lax.axis_index("c")
        sid = lax.axis_index("s")
        wid = sid * NC + cid
        pltpu.sync_copy(dst_hbm.at[wid], dst_v)
        pltpu.sync_copy(ones_hbm, ones_v)
        r0 = sid * ROWS_PER_TILE
        pltpu.sync_copy(zeros_hbm.at[pl.ds(r0, ROWS_PER_TILE)],
                        acc_sh.at[pl.ds(r0, ROWS_PER_TILE)])
        plsc.subcore_barrier()

        def body(j, carry):
            pltpu.sync_copy(ones_v, acc_sh.at[dst_v.at[j]], add=True)
            return carry

        lax.fori_loop(0, CPW, body, 0)
        plsc.subcore_barrier()
        pltpu.sync_copy(acc_sh.at[pl.ds(r0, ROWS_PER_TILE)],
                        out_hbm.at[cid, pl.ds(r0, ROWS_PER_TILE)])

    return deg_kernel(dst_r, ones_col, zeros_col)


def _sc_edge_agg(table, src_r, dst_r, zeros_tab, feat, stage_table=False):
    scratch = [
        pltpu.VMEM((CPW, CHUNK), jnp.int32),
        pltpu.VMEM((CPW, CHUNK), jnp.int32),
        [pltpu.VMEM((CHUNK, feat), jnp.float32) for _ in range(NBUF)],
        [pltpu.SemaphoreType.DMA for _ in range(NBUF)],
        [pltpu.SemaphoreType.DMA for _ in range(NBUF)],
        pltpu.VMEM_SHARED((N_PAD, feat), jnp.float32),
    ]
    if stage_table:
        scratch.append(pltpu.VMEM_SHARED((N_PAD, feat), jnp.float32))

    @functools.partial(
        pl.kernel,
        out_type=jax.ShapeDtypeStruct((NC, N_PAD, feat), jnp.float32),
        mesh=_mesh(),
        compiler_params=_SC_PARAMS,
        scratch_types=scratch,
    )
    def agg_kernel(tab_hbm, src_hbm, dst_hbm, zeros_hbm, out_hbm,
                   src_v, dst_v, bufs, gsems, ssems, acc_sh, *maybe_tab):
        tab_ref = maybe_tab[0] if stage_table else tab_hbm
        cid = lax.axis_index("c")
        sid = lax.axis_index("s")
        wid = sid * NC + cid
        pltpu.sync_copy(src_hbm.at[wid], src_v)
        pltpu.sync_copy(dst_hbm.at[wid], dst_v)
        r0 = sid * ROWS_PER_TILE

        if stage_table:
            pltpu.sync_copy(tab_hbm.at[pl.ds(r0, ROWS_PER_TILE)],
                            tab_ref.at[pl.ds(r0, ROWS_PER_TILE)])

        @pl.when(cid == 0)
        def _():
            pltpu.sync_copy(tab_hbm.at[pl.ds(r0, ROWS_PER_TILE)],
                            acc_sh.at[pl.ds(r0, ROWS_PER_TILE)])

        @pl.when(cid != 0)
        def _():
            pltpu.sync_copy(zeros_hbm.at[pl.ds(r0, ROWS_PER_TILE)],
                            acc_sh.at[pl.ds(r0, ROWS_PER_TILE)])

        plsc.subcore_barrier()

        def wait_gather(b):
            pltpu.make_async_copy(tab_hbm.at[pl.ds(0, CHUNK)], bufs[b],
                                  gsems[b]).wait()

        def wait_scatter(b):
            pltpu.make_async_copy(bufs[b], acc_sh.at[dst_v.at[0]],
                                  ssems[b]).wait()

        def body(t, carry):
            j0 = NBUF * t
            for b in range(NBUF):
                @pl.when(t > 0)
                def _(b=b):
                    wait_scatter(b)
                pltpu.async_copy(tab_ref.at[src_v.at[j0 + b]], bufs[b],
                                 gsems[b])
            for b in range(NBUF):
                wait_gather(b)
                pltpu.async_copy(bufs[b], acc_sh.at[dst_v.at[j0 + b]],
                                 ssems[b], add=True)
            return carry

        lax.fori_loop(0, CPW // NBUF, body, 0)
        for b in range(NBUF):
            wait_scatter(b)
        plsc.subcore_barrier()
        pltpu.sync_copy(acc_sh.at[pl.ds(r0, ROWS_PER_TILE)],
                        out_hbm.at[cid, pl.ds(r0, ROWS_PER_TILE)])

    return agg_kernel(table, src_r, dst_r, zeros_tab)


def _sc_edge_agg_halves(tab_a, tab_b, src_r2, dst_r2):
    @functools.partial(
        pl.kernel,
        out_type=jax.ShapeDtypeStruct((NC, N_PAD, 32), jnp.float32),
        mesh=_mesh(),
        compiler_params=_SC_PARAMS,
        scratch_types=[
            pltpu.VMEM((CPT2, CHUNK), jnp.int32),
            pltpu.VMEM((CPT2, CHUNK), jnp.int32),
            [pltpu.VMEM((CHUNK, 32), jnp.float32) for _ in range(NBUF)],
            [pltpu.SemaphoreType.DMA for _ in range(NBUF)],
            [pltpu.SemaphoreType.DMA for _ in range(NBUF)],
            pltpu.VMEM_SHARED((N_PAD, 32), jnp.float32),
            pltpu.VMEM_SHARED((N_PAD, 32), jnp.float32),
        ],
    )
    def agg_kernel(taba_hbm, tabb_hbm, src_hbm, dst_hbm, out_hbm,
                   src_v, dst_v, bufs, gsems, ssems, acc_sh, tab_sh):
        cid = lax.axis_index("c")
        sid = lax.axis_index("s")
        pltpu.sync_copy(src_hbm.at[sid], src_v)
        pltpu.sync_copy(dst_hbm.at[sid], dst_v)
        r0 = sid * ROWS_PER_TILE

        @pl.when(cid == 0)
        def _():
            pltpu.sync_copy(taba_hbm.at[pl.ds(r0, ROWS_PER_TILE)],
                            tab_sh.at[pl.ds(r0, ROWS_PER_TILE)])
            pltpu.sync_copy(taba_hbm.at[pl.ds(r0, ROWS_PER_TILE)],
                            acc_sh.at[pl.ds(r0, ROWS_PER_TILE)])

        @pl.when(cid != 0)
        def _():
            pltpu.sync_copy(tabb_hbm.at[pl.ds(r0, ROWS_PER_TILE)],
                            tab_sh.at[pl.ds(r0, ROWS_PER_TILE)])
            pltpu.sync_copy(tabb_hbm.at[pl.ds(r0, ROWS_PER_TILE)],
                            acc_sh.at[pl.ds(r0, ROWS_PER_TILE)])

        plsc.subcore_barrier()

        def wait_gather(b):
            pltpu.make_async_copy(taba_hbm.at[pl.ds(0, CHUNK)], bufs[b],
                                  gsems[b]).wait()

        def wait_scatter(b):
            pltpu.make_async_copy(bufs[b], acc_sh.at[dst_v.at[0]],
                                  ssems[b]).wait()

        def body(t, carry):
            j0 = NBUF * t
            for b in range(NBUF):
                @pl.when(t > 0)
                def _(b=b):
                    wait_scatter(b)
                pltpu.async_copy(tab_sh.at[src_v.at[j0 + b]], bufs[b],
                                 gsems[b])
            for b in range(NBUF):
                wait_gather(b)
                pltpu.async_copy(bufs[b], acc_sh.at[dst_v.at[j0 + b]],
                                 ssems[b], add=True)
            return carry

        lax.fori_loop(0, CPT2 // NBUF, body, 0)
        for b in range(NBUF):
            wait_scatter(b)
        plsc.subcore_barrier()
        pltpu.sync_copy(acc_sh.at[pl.ds(r0, ROWS_PER_TILE)],
                        out_hbm.at[cid, pl.ds(r0, ROWS_PER_TILE)])

    return agg_kernel(tab_a, tab_b, src_r2, dst_r2)


def _tc_dense1(x_pad, deg_ab, w1):
    def body(x_ref, deg_ref, w_ref, scaled_ref, dinv_ref):
        deg = deg_ref[0, :, :1] + deg_ref[1, :, :1] + 1.0
        dinv = lax.rsqrt(deg)
        h = jnp.dot(x_ref[...], w_ref[...], preferred_element_type=jnp.float32)
        scaled_ref[...] = h * dinv
        dinv_ref[...] = dinv

    return pl.pallas_call(
        body,
        out_shape=(jax.ShapeDtypeStruct((N_PAD, 32), jnp.float32),
                   jax.ShapeDtypeStruct((N_PAD, 1), jnp.float32)),
    )(x_pad, deg_ab, w1)


def _tc_dense2(agg_ab, dinv, b1, w2):
    def body(agg_ref, dinv_ref, b_ref, w_ref, outa_ref, outb_ref):
        dinv = dinv_ref[...]
        tot = agg_ref[0] + agg_ref[1]
        out1 = jnp.maximum(tot * dinv + b_ref[...], 0.0)
        h2 = jnp.dot(out1, w_ref[...], preferred_element_type=jnp.float32)
        scaled2 = h2 * dinv
        outa_ref[...] = scaled2[:, :32]
        outb_ref[...] = scaled2[:, 32:]

    return pl.pallas_call(
        body,
        out_shape=(jax.ShapeDtypeStruct((N_PAD, 32), jnp.float32),
                   jax.ShapeDtypeStruct((N_PAD, 32), jnp.float32)),
    )(agg_ab, dinv, b1, w2)


def _tc_dense3(agg2, dinv, b2):
    def body(agg_ref, dinv_ref, b_ref, out_ref):
        tot = jnp.concatenate([agg_ref[0], agg_ref[1]], axis=1)
        out_ref[...] = jnp.maximum(tot * dinv_ref[...] + b_ref[...], 0.0)

    return pl.pallas_call(
        body,
        out_shape=jax.ShapeDtypeStruct((N_PAD, 64), jnp.float32),
    )(agg2, dinv, b2)


def _tc_heads(x3, wtop, wbot, bf1, wf2, bf2):
    def body(x3_ref, wtop_ref, wbot_ref, bf1_ref, wf2_ref, bf2_ref, out_ref):
        x3 = x3_ref[...]
        ctx = x3[:, 0:64]
        for k in range(1, NPG):
            ctx = ctx + x3[:, 64 * k:64 * k + 64]
        ctx = ctx * (1.0 / NPG)
        ctx_part = jnp.dot(ctx, wbot_ref[...], preferred_element_type=jnp.float32)
        qs = []
        for node in (0, 8):
            s = x3[:, 64 * node:64 * node + 64]
            pre = jnp.dot(s, wtop_ref[...], preferred_element_type=jnp.float32)
            pre = jnp.maximum(pre + ctx_part + bf1_ref[...], 0.0)
            q = jnp.dot(pre, wf2_ref[...], preferred_element_type=jnp.float32)
            qs.append(q + bf2_ref[...])
        out_ref[...] = jnp.concatenate(qs, axis=1)

    rows = x3.shape[0]
    return pl.pallas_call(
        body,
        out_shape=jax.ShapeDtypeStruct((rows, 2), jnp.float32),
    )(x3, wtop, wbot, bf1, wf2, bf2)


def kernel(x, edge_index, batch, W1, b1, W2, b2, Wf1, bf1, Wf2, bf2):
    del batch

    src = edge_index[0].astype(jnp.int32)
    dst = edge_index[1].astype(jnp.int32)
    n_edges = src.shape[0]
    pad = E_PAD - n_edges
    src_p = jnp.concatenate([src, jnp.zeros((pad,), jnp.int32)])
    dst_p = jnp.concatenate([dst, jnp.full((pad,), N_NODES, jnp.int32)])
    src_r = src_p.reshape(NW, CPW, CHUNK)
    dst_r = dst_p.reshape(NW, CPW, CHUNK)
    src_r2 = src_p.reshape(NS, CPT2, CHUNK)
    dst_r2 = dst_p.reshape(NS, CPT2, CHUNK)

    x_pad = jnp.concatenate([x, jnp.zeros((N_PAD - N_NODES, x.shape[1]), x.dtype)])
    ones_col = jnp.ones((CHUNK, DEG_F), jnp.float32)
    zeros_col = jnp.zeros((N_PAD, DEG_F), jnp.float32)
    zeros32 = jnp.zeros((N_PAD, 32), jnp.float32)

    deg_ab = _sc_degree(dst_r, ones_col, zeros_col)
    scaled1, dinv = _tc_dense1(x_pad, deg_ab, W1)

    agg1 = _sc_edge_agg(scaled1, src_r, dst_r, zeros32, 32, stage_table=True)
    scaled2a, scaled2b = _tc_dense2(agg1, dinv, b1.reshape(1, 32), W2)

    agg2 = _sc_edge_agg_halves(scaled2a, scaled2b, src_r2, dst_r2)
    out2 = _tc_dense3(agg2, dinv, b2.reshape(1, 64))

    x3 = out2[:N_NODES].reshape(NUM_GRAPHS, NPG * 64)
    x3 = jnp.concatenate([x3, jnp.zeros((1, NPG * 64), jnp.float32)])
    q = _tc_heads(x3, Wf1[:64], Wf1[64:], bf1.reshape(1, 64), Wf2,
                  bf2.reshape(1, 1))
    return q[:NUM_GRAPHS]


# speedup vs baseline: 1.7848x; 1.0098x over previous
import functools

import jax
import jax.numpy as jnp
from jax import lax
from jax.experimental import pallas as pl
from jax.experimental.pallas import tpu as pltpu
from jax.experimental.pallas import tpu_sc as plsc

N_NODES = 9999
N_PAD = 10112
NUM_GRAPHS = 1111
NPG = 9
NC = 2
NS = 16
NW = NC * NS
CHUNK = 256
CPW = 40
NBUF = 4
EPW = CPW * CHUNK
E_PAD = NW * EPW
CPT2 = CPW * NC
ROWS_PER_TILE = N_PAD // NS
DEG_F = 8


def _mesh():
    return plsc.VectorSubcoreMesh(core_axis_name="c", subcore_axis_name="s")


_SC_PARAMS = pltpu.CompilerParams(use_tc_tiling_on_sc=False)


def _sc_degree(dst_r, ones_col, zeros_col):
    @functools.partial(
        pl.kernel,
        out_type=jax.ShapeDtypeStruct((NC, N_PAD, DEG_F), jnp.float32),
        mesh=_mesh(),
        compiler_params=_SC_PARAMS,
        scratch_types=[
            pltpu.VMEM((CPW, CHUNK), jnp.int32),
            pltpu.VMEM((CHUNK, DEG_F), jnp.float32),
            pltpu.VMEM_SHARED((N_PAD, DEG_F), jnp.float32),
        ],
    )
    def deg_kernel(dst_hbm, ones_hbm, zeros_hbm, out_hbm, dst_v, ones_v, acc_sh):
        cid = lax.axis_index("c")
        sid = lax.axis_index("s")
        wid = sid * NC + cid
        pltpu.sync_copy(dst_hbm.at[wid], dst_v)
        pltpu.sync_copy(ones_hbm, ones_v)
        r0 = sid * ROWS_PER_TILE
        pltpu.sync_copy(zeros_hbm.at[pl.ds(r0, ROWS_PER_TILE)],
                        acc_sh.at[pl.ds(r0, ROWS_PER_TILE)])
        plsc.subcore_barrier()

        def body(j, carry):
            pltpu.sync_copy(ones_v, acc_sh.at[dst_v.at[j]], add=True)
            return carry

        lax.fori_loop(0, CPW, body, 0)
        plsc.subcore_barrier()
        pltpu.sync_copy(acc_sh.at[pl.ds(r0, ROWS_PER_TILE)],
                        out_hbm.at[cid, pl.ds(r0, ROWS_PER_TILE)])

    return deg_kernel(dst_r, ones_col, zeros_col)


def _sc_edge_agg(table, src_r, dst_r, zeros_tab, feat, stage_table=False):
    scratch = [
        pltpu.VMEM((CPW, CHUNK), jnp.int32),
        pltpu.VMEM((CPW, CHUNK), jnp.int32),
        [pltpu.VMEM((CHUNK, feat), jnp.float32) for _ in range(NBUF)],
        [pltpu.SemaphoreType.DMA for _ in range(NBUF)],
        [pltpu.SemaphoreType.DMA for _ in range(NBUF)],
        pltpu.VMEM_SHARED((N_PAD, feat), jnp.float32),
    ]
    if stage_table:
        scratch.append(pltpu.VMEM_SHARED((N_PAD, feat), jnp.float32))

    @functools.partial(
        pl.kernel,
        out_type=jax.ShapeDtypeStruct((NC, N_PAD, feat), jnp.float32),
        mesh=_mesh(),
        compiler_params=_SC_PARAMS,
        scratch_types=scratch,
    )
    def agg_kernel(tab_hbm, src_hbm, dst_hbm, zeros_hbm, out_hbm,
                   src_v, dst_v, bufs, gsems, ssems, acc_sh, *maybe_tab):
        tab_ref = maybe_tab[0] if stage_table else tab_hbm
        cid = lax.axis_index("c")
        sid = lax.axis_index("s")
        wid = sid * NC + cid
        pltpu.sync_copy(src_hbm.at[wid], src_v)
        pltpu.sync_copy(dst_hbm.at[wid], dst_v)
        r0 = sid * ROWS_PER_TILE

        if stage_table:
            pltpu.sync_copy(tab_hbm.at[pl.ds(r0, ROWS_PER_TILE)],
                            tab_ref.at[pl.ds(r0, ROWS_PER_TILE)])

        @pl.when(cid == 0)
        def _():
            pltpu.sync_copy(tab_hbm.at[pl.ds(r0, ROWS_PER_TILE)],
                            acc_sh.at[pl.ds(r0, ROWS_PER_TILE)])

        @pl.when(cid != 0)
        def _():
            pltpu.sync_copy(zeros_hbm.at[pl.ds(r0, ROWS_PER_TILE)],
                            acc_sh.at[pl.ds(r0, ROWS_PER_TILE)])

        plsc.subcore_barrier()

        def wait_gather(b):
            pltpu.make_async_copy(tab_hbm.at[pl.ds(0, CHUNK)], bufs[b],
                                  gsems[b]).wait()

        def wait_scatter(b):
            pltpu.make_async_copy(bufs[b], acc_sh.at[dst_v.at[0]],
                                  ssems[b]).wait()

        def body(t, carry):
            j0 = NBUF * t
            for b in range(NBUF):
                @pl.when(t > 0)
                def _(b=b):
                    wait_scatter(b)
                pltpu.async_copy(tab_ref.at[src_v.at[j0 + b]], bufs[b],
                                 gsems[b])
            for b in range(NBUF):
                wait_gather(b)
                pltpu.async_copy(bufs[b], acc_sh.at[dst_v.at[j0 + b]],
                                 ssems[b], add=True)
            return carry

        lax.fori_loop(0, CPW // NBUF, body, 0)
        for b in range(NBUF):
            wait_scatter(b)
        plsc.subcore_barrier()
        pltpu.sync_copy(acc_sh.at[pl.ds(r0, ROWS_PER_TILE)],
                        out_hbm.at[cid, pl.ds(r0, ROWS_PER_TILE)])

    return agg_kernel(table, src_r, dst_r, zeros_tab)


def _sc_edge_agg_halves(tab_a, tab_b, src_r2, dst_r2):
    @functools.partial(
        pl.kernel,
        out_type=jax.ShapeDtypeStruct((NC, N_PAD, 32), jnp.float32),
        mesh=_mesh(),
        compiler_params=_SC_PARAMS,
        scratch_types=[
            pltpu.VMEM((CPT2, CHUNK), jnp.int32),
            pltpu.VMEM((CPT2, CHUNK), jnp.int32),
            [pltpu.VMEM((CHUNK, 32), jnp.float32) for _ in range(NBUF)],
            [pltpu.SemaphoreType.DMA for _ in range(NBUF)],
            [pltpu.SemaphoreType.DMA for _ in range(NBUF)],
            pltpu.VMEM_SHARED((N_PAD, 32), jnp.float32),
            pltpu.VMEM_SHARED((N_PAD, 32), jnp.float32),
        ],
    )
    def agg_kernel(taba_hbm, tabb_hbm, src_hbm, dst_hbm, out_hbm,
                   src_v, dst_v, bufs, gsems, ssems, acc_sh, tab_sh):
        cid = lax.axis_index("c")
        sid = lax.axis_index("s")
        pltpu.sync_copy(src_hbm.at[sid], src_v)
        pltpu.sync_copy(dst_hbm.at[sid], dst_v)
        r0 = sid * ROWS_PER_TILE

        @pl.when(cid == 0)
        def _():
            pltpu.sync_copy(taba_hbm.at[pl.ds(r0, ROWS_PER_TILE)],
                            tab_sh.at[pl.ds(r0, ROWS_PER_TILE)])
            pltpu.sync_copy(taba_hbm.at[pl.ds(r0, ROWS_PER_TILE)],
                            acc_sh.at[pl.ds(r0, ROWS_PER_TILE)])

        @pl.when(cid != 0)
        def _():
            pltpu.sync_copy(tabb_hbm.at[pl.ds(r0, ROWS_PER_TILE)],
                            tab_sh.at[pl.ds(r0, ROWS_PER_TILE)])
            pltpu.sync_copy(tabb_hbm.at[pl.ds(r0, ROWS_PER_TILE)],
                            acc_sh.at[pl.ds(r0, ROWS_PER_TILE)])

        plsc.subcore_barrier()

        def wait_gather(b):
            pltpu.make_async_copy(taba_hbm.at[pl.ds(0, CHUNK)], bufs[b],
                                  gsems[b]).wait()

        def wait_scatter(b):
            pltpu.make_async_copy(bufs[b], acc_sh.at[dst_v.at[0]],
                                  ssems[b]).wait()

        def body(t, carry):
            j0 = NBUF * t
            for b in range(NBUF):
                @pl.when(t > 0)
                def _(b=b):
                    wait_scatter(b)
                pltpu.async_copy(tab_sh.at[src_v.at[j0 + b]], bufs[b],
                                 gsems[b])
            for b in range(NBUF):
                wait_gather(b)
                pltpu.async_copy(bufs[b], acc_sh.at[dst_v.at[j0 + b]],
                                 ssems[b], add=True)
            return carry

        lax.fori_loop(0, CPT2 // NBUF, body, 0)
        for b in range(NBUF):
            wait_scatter(b)
        plsc.subcore_barrier()
        pltpu.sync_copy(acc_sh.at[pl.ds(r0, ROWS_PER_TILE)],
                        out_hbm.at[cid, pl.ds(r0, ROWS_PER_TILE)])

    return agg_kernel(tab_a, tab_b, src_r2, dst_r2)


def _tc_dense1(x_pad, deg_ab, w1):
    def body(x_ref, deg_ref, w_ref, scaled_ref, dinv_ref):
        deg = deg_ref[0, :, :1] + deg_ref[1, :, :1] + 1.0
        dinv = lax.rsqrt(deg)
        h = jnp.dot(x_ref[...], w_ref[...], preferred_element_type=jnp.float32)
        scaled_ref[...] = h * dinv
        dinv_ref[...] = dinv

    return pl.pallas_call(
        body,
        out_shape=(jax.ShapeDtypeStruct((N_PAD, 32), jnp.float32),
                   jax.ShapeDtypeStruct((N_PAD, 1), jnp.float32)),
    )(x_pad, deg_ab, w1)


def _tc_dense2(agg_ab, dinv, b1, w2):
    def body(agg_ref, dinv_ref, b_ref, w_ref, outa_ref, outb_ref):
        dinv = dinv_ref[...]
        tot = agg_ref[0] + agg_ref[1]
        out1 = jnp.maximum(tot * dinv + b_ref[...], 0.0)
        h2 = jnp.dot(out1, w_ref[...], preferred_element_type=jnp.float32)
        scaled2 = h2 * dinv
        outa_ref[...] = scaled2[:, :32]
        outb_ref[...] = scaled2[:, 32:]

    return pl.pallas_call(
        body,
        out_shape=(jax.ShapeDtypeStruct((N_PAD, 32), jnp.float32),
                   jax.ShapeDtypeStruct((N_PAD, 32), jnp.float32)),
    )(agg_ab, dinv, b1, w2)


def _tc_dense3(agg2, dinv, b2):
    def body(agg_ref, dinv_ref, b_ref, out_ref):
        tot = jnp.concatenate([agg_ref[0], agg_ref[1]], axis=1)
        out_ref[...] = jnp.maximum(tot * dinv_ref[...] + b_ref[...], 0.0)

    return pl.pallas_call(
        body,
        out_shape=jax.ShapeDtypeStruct((N_PAD, 64), jnp.float32),
    )(agg2, dinv, b2)


def _tc_heads(x3, wtop, wbot, bf1, wf2, bf2):
    def body(x3_ref, wtop_ref, wbot_ref, bf1_ref, wf2_ref, bf2_ref, out_ref):
        x3 = x3_ref[...]
        ctx = x3[:, 0:64]
        for k in range(1, NPG):
            ctx = ctx + x3[:, 64 * k:64 * k + 64]
        ctx = ctx * (1.0 / NPG)
        ctx_part = jnp.dot(ctx, wbot_ref[...], preferred_element_type=jnp.float32)
        qs = []
        for node in (0, 8):
            s = x3[:, 64 * node:64 * node + 64]
            pre = jnp.dot(s, wtop_ref[...], preferred_element_type=jnp.float32)
            pre = jnp.maximum(pre + ctx_part + bf1_ref[...], 0.0)
            q = jnp.dot(pre, wf2_ref[...], preferred_element_type=jnp.float32)
            qs.append(q + bf2_ref[...])
        out_ref[...] = jnp.concatenate(qs, axis=1)

    rows = x3.shape[0]
    return pl.pallas_call(
        body,
        out_shape=jax.ShapeDtypeStruct((rows, 2), jnp.float32),
    )(x3, wtop, wbot, bf1, wf2, bf2)


def kernel(x, edge_index, batch, W1, b1, W2, b2, Wf1, bf1, Wf2, bf2):
    del batch

    src = edge_index[0].astype(jnp.int32)
    dst = edge_index[1].astype(jnp.int32)
    n_edges = src.shape[0]
    pad = E_PAD - n_edges
    src_p = jnp.concatenate([src, jnp.zeros((pad,), jnp.int32)])
    dst_p = jnp.concatenate([dst, jnp.full((pad,), N_NODES, jnp.int32)])
    src_r = src_p.reshape(NW, CPW, CHUNK)
    dst_r = dst_p.reshape(NW, CPW, CHUNK)
    src_r2 = src_p.reshape(NS, CPT2, CHUNK)
    dst_r2 = dst_p.reshape(NS, CPT2, CHUNK)

    x_pad = jnp.concatenate([x, jnp.zeros((N_PAD - N_NODES, x.shape[1]), x.dtype)])
    ones_col = jnp.ones((CHUNK, DEG_F), jnp.float32)
    zeros_col = jnp.zeros((N_PAD, DEG_F), jnp.float32)
    zeros32 = jnp.zeros((N_PAD, 32), jnp.float32)

    deg_ab = _sc_degree(dst_r, ones_col, zeros_col)
    scaled1, dinv = _tc_dense1(x_pad, deg_ab, W1)

    agg1 = _sc_edge_agg(scaled1, src_r, dst_r, zeros32, 32, stage_table=True)
    scaled2a, scaled2b = _tc_dense2(agg1, dinv, b1.reshape(1, 32), W2)

    agg2 = _sc_edge_agg_halves(scaled2a, scaled2b, src_r2, dst_r2)
    out2 = _tc_dense3(agg2, dinv, b2.reshape(1, 64))

    x3 = out2[:N_NODES].reshape(NUM_GRAPHS, NPG * 64)
    x3 = jnp.concatenate([x3, jnp.zeros((1, NPG * 64), jnp.float32)])
    q = _tc_heads(x3, Wf1[:64], Wf1[64:], bf1.reshape(1, 64), Wf2,
                  bf2.reshape(1, 1))
    return q[:NUM_GRAPHS]
